# baseline (speedup 1.0000x reference)
import jax
import jax.numpy as jnp
from jax import lax
from jax.experimental import pallas as pl
from jax.experimental.pallas import tpu as pltpu
from jax.experimental.pallas import tpu_sc as plsc

NC, NS, L = 2, 16, 16
NW = NC * NS
B = 1024
N = 200
NPAD = 208
GPB = NPAD // L
BPW = B // NW
D = 128
ED = 2 * D
RD = 3 * D
CHUNK = 64
GPC = CHUNK // L
ROWS_PW = BPW * NPAD
NCHUNK = ROWS_PW // CHUNK
GAMMA = 12.0


def _rsqrt16(x):
    xi = plsc.bitcast(x, jnp.int32)
    yi = jnp.int32(0x5F3759DF) - lax.shift_right_logical(xi, 1)
    y = plsc.bitcast(yi, jnp.float32)
    for _ in range(3):
        y = y * (1.5 - 0.5 * x * y * y)
    return y


def _body(ent_hbm, rel_hbm, relidx_hbm, tailidx_hbm, neg_hbm, out_hbm,
          tidx_v, ridx_v, tail_v, rel_v, c1_v, c2_v, c3_v, nidx_v, out_v,
          rb0, rb1, sem0, sem1, gsem):
    wid = lax.axis_index("s") * NC + lax.axis_index("c")
    b0 = pl.multiple_of(wid * BPW, BPW)
    lanes = lax.iota(jnp.int32, L)
    zf = jnp.zeros((L,), jnp.float32)

    pltpu.sync_copy(tailidx_hbm.at[pl.ds(b0, BPW)], tidx_v)
    pltpu.sync_copy(relidx_hbm.at[pl.ds(b0, BPW)], ridx_v)
    pltpu.make_async_copy(ent_hbm.at[tidx_v], tail_v, gsem).start()
    pltpu.make_async_copy(ent_hbm.at[tidx_v], tail_v, gsem).wait()
    pltpu.make_async_copy(rel_hbm.at[ridx_v], rel_v, gsem).start()
    pltpu.make_async_copy(rel_hbm.at[ridx_v], rel_v, gsem).wait()

    zi = jnp.zeros((L,), jnp.int32)
    for b in range(BPW):
        nidx_v[pl.ds(b * NPAD + NPAD - L, L)] = zi
        pltpu.sync_copy(neg_hbm.at[b0 + b], nidx_v.at[pl.ds(b * NPAD, N)])

    for gb in range(BPW // L):
        rows = jnp.int32(gb * L) + lanes

        def ssq_body(d, carry):
            aa, ab = carry
            dv = jnp.full((L,), d, jnp.int32)
            va = plsc.load_gather(tail_v, [rows, dv])
            vb = plsc.load_gather(tail_v, [rows, dv + D])
            return (aa + va * va, ab + vb * vb)

        aa, ab = lax.fori_loop(0, D, ssq_body, (zf, zf))
        ra = _rsqrt16(aa)
        rb = _rsqrt16(ab)

        def cstore_body(d, _):
            dv = jnp.full((L,), d, jnp.int32)
            va = plsc.load_gather(tail_v, [rows, dv])
            vb = plsc.load_gather(tail_v, [rows, dv + D])
            vm = plsc.load_gather(rel_v, [rows, dv + D])
            c2 = va * ra
            plsc.store_scatter(c1_v, [rows, dv], vb * rb + 1.0)
            plsc.store_scatter(c2_v, [rows, dv], c2)
            plsc.store_scatter(c3_v, [rows, dv], vm - c2)
            return 0

        lax.fori_loop(0, D, cstore_body, 0)

    def gather_chunk(c, buf, sem):
        off = pl.multiple_of(c * CHUNK, CHUNK)
        return pltpu.make_async_copy(
            ent_hbm.at[nidx_v.at[pl.ds(off, CHUNK)]], buf, sem)

    gather_chunk(0, rb0, sem0).start()
    gather_chunk(1, rb1, sem1).start()

    def process_group(buf, j, b, gidx):
        rows = jnp.int32(j * L) + lanes

        def ssq_body(d, carry):
            aa, ab = carry
            dv = jnp.full((L,), d, jnp.int32)
            va = plsc.load_gather(buf, [rows, dv])
            vb = plsc.load_gather(buf, [rows, dv + D])
            return (aa + va * va, ab + vb * vb)

        aa, ab = lax.fori_loop(0, D, ssq_body, (zf, zf))
        ra = _rsqrt16(aa)
        rb = _rsqrt16(ab)

        def score_chunk(k, acc):
            base = pl.multiple_of(k * L, L)
            c1c = c1_v[b, pl.ds(base, L)]
            c2c = c2_v[b, pl.ds(base, L)]
            c3c = c3_v[b, pl.ds(base, L)]
            for j in range(L):
                dv = jnp.full((L,), k * L + j, jnp.int32)
                va = plsc.load_gather(buf, [rows, dv])
                vb = plsc.load_gather(buf, [rows, dv + D])
                t = va * ra * c1c[j] - vb * rb * c2c[j] + c3c[j]
                acc = acc + jnp.abs(t)
            return acc

        acc = lax.fori_loop(0, D // L, score_chunk, zf)
        out_v[pl.ds(pl.multiple_of(gidx * L, L), L)] = GAMMA - acc

    def chunk_pair(i, carry):
        b, g = carry
        for half, (buf, sem) in enumerate(((rb0, sem0), (rb1, sem1))):
            cc = i * 2 + half
            gather_chunk(cc, buf, sem).wait()
            for j in range(GPC):
                process_group(buf, j, b, cc * GPC + j)
                g = g + 1
                wrap = (g >= GPB).astype(jnp.int32)
                g = g * (1 - wrap)
                b = b + wrap
            nc = cc + 2

            @pl.when(nc < NCHUNK)
            def _():
                gather_chunk(nc, buf, sem).start()
        return (b, g)

    lax.fori_loop(0, NCHUNK // 2, chunk_pair,
                  (jnp.int32(0), jnp.int32(0)))

    for b in range(BPW):
        pltpu.sync_copy(out_v.at[pl.ds(b * NPAD, N)], out_hbm.at[b0 + b])


def kernel(entity_embedding, relation_embedding, positive_sample, negative_sample):
    rel_idx = positive_sample[:, 1].astype(jnp.int32)
    tail_idx = positive_sample[:, 2].astype(jnp.int32)
    mesh = plsc.VectorSubcoreMesh(core_axis_name="c", subcore_axis_name="s",
                                  num_cores=NC, num_subcores=NS)
    f = pl.kernel(
        _body,
        out_type=jax.ShapeDtypeStruct((B, N), jnp.float32),
        mesh=mesh,
        compiler_params=pltpu.CompilerParams(use_tc_tiling_on_sc=False,
                                             needs_layout_passes=False),
        scratch_types=[
            pltpu.VMEM((BPW,), jnp.int32),
            pltpu.VMEM((BPW,), jnp.int32),
            pltpu.VMEM((BPW, ED), jnp.float32),
            pltpu.VMEM((BPW, RD), jnp.float32),
            pltpu.VMEM((BPW, D), jnp.float32),
            pltpu.VMEM((BPW, D), jnp.float32),
            pltpu.VMEM((BPW, D), jnp.float32),
            pltpu.VMEM((ROWS_PW,), jnp.int32),
            pltpu.VMEM((ROWS_PW,), jnp.float32),
            pltpu.VMEM((CHUNK, ED), jnp.float32),
            pltpu.VMEM((CHUNK, ED), jnp.float32),
            pltpu.SemaphoreType.DMA,
            pltpu.SemaphoreType.DMA,
            pltpu.SemaphoreType.DMA,
        ],
    )
    return f(entity_embedding, relation_embedding, rel_idx, tail_idx,
             negative_sample)


# speedup vs baseline: 3.3990x; 3.3990x over previous
import jax
import jax.numpy as jnp
from jax import lax
from jax.experimental import pallas as pl
from jax.experimental.pallas import tpu as pltpu
from jax.experimental.pallas import tpu_sc as plsc

NC, NS, L = 2, 16, 16
NW = NC * NS
B = 1024
N = 200
NPAD = 208
GPB = NPAD // L
BPW = B // NW
D = 128
ED = 2 * D
RD = 3 * D
CHUNK = 64
GPC = CHUNK // L
ROWS_PW = BPW * NPAD
NCHUNK = ROWS_PW // CHUNK
GAMMA = 12.0


def _rsqrt16(x):
    xi = plsc.bitcast(x, jnp.int32)
    yi = jnp.int32(0x5F3759DF) - lax.shift_right_logical(xi, 1)
    y = plsc.bitcast(yi, jnp.float32)
    for _ in range(3):
        y = y * (1.5 - 0.5 * x * y * y)
    return y


def _body(ent_hbm, rel_hbm, relidx_hbm, tailidx_hbm, neg_hbm, out_hbm,
          tidx_v, ridx_v, tail_v, rel_v, c1_v, c2_v, c3_v, nidx_v, out_v,
          rb0, rb1, sem0, sem1, gsem):
    wid = lax.axis_index("s") * NC + lax.axis_index("c")
    b0 = pl.multiple_of(wid * BPW, BPW)
    lanes = lax.iota(jnp.int32, L)
    zf = jnp.zeros((L,), jnp.float32)

    pltpu.sync_copy(tailidx_hbm.at[pl.ds(b0, BPW)], tidx_v)
    pltpu.sync_copy(relidx_hbm.at[pl.ds(b0, BPW)], ridx_v)
    pltpu.make_async_copy(ent_hbm.at[tidx_v], tail_v, gsem).start()
    pltpu.make_async_copy(ent_hbm.at[tidx_v], tail_v, gsem).wait()
    pltpu.make_async_copy(rel_hbm.at[ridx_v], rel_v, gsem).start()
    pltpu.make_async_copy(rel_hbm.at[ridx_v], rel_v, gsem).wait()

    zi = jnp.zeros((L,), jnp.int32)
    for b in range(BPW):
        nidx_v[pl.ds(b * NPAD + NPAD - L, L)] = zi
        pltpu.sync_copy(neg_hbm.at[b0 + b], nidx_v.at[pl.ds(b * NPAD, N)])

    for gb in range(BPW // L):
        rows = jnp.int32(gb * L) + lanes

        def ssq_body(d, carry):
            aa, ab = carry
            dv = jnp.full((L,), d, jnp.int32)
            va = plsc.load_gather(tail_v, [rows, dv])
            vb = plsc.load_gather(tail_v, [rows, dv + D])
            return (aa + va * va, ab + vb * vb)

        aa, ab = lax.fori_loop(0, D, ssq_body, (zf, zf))
        ra = _rsqrt16(aa)
        rb = _rsqrt16(ab)

        def cstore_body(d, _):
            dv = jnp.full((L,), d, jnp.int32)
            va = plsc.load_gather(tail_v, [rows, dv])
            vb = plsc.load_gather(tail_v, [rows, dv + D])
            vm = plsc.load_gather(rel_v, [rows, dv + D])
            c2 = va * ra
            plsc.store_scatter(c1_v, [rows, dv], vb * rb + 1.0)
            plsc.store_scatter(c2_v, [rows, dv], c2)
            plsc.store_scatter(c3_v, [rows, dv], vm - c2)
            return 0

        lax.fori_loop(0, D, cstore_body, 0)

    def gather_chunk(c, buf, sem):
        off = pl.multiple_of(c * CHUNK, CHUNK)
        return pltpu.make_async_copy(
            ent_hbm.at[nidx_v.at[pl.ds(off, CHUNK)]], buf, sem)

    gather_chunk(0, rb0, sem0).start()
    gather_chunk(1, rb1, sem1).start()

    HALF = D // L

    def process_group(buf, j, b, gidx):
        c1c = [c1_v[b, pl.ds(k * L, L)] for k in range(HALF)]
        c2c = [c2_v[b, pl.ds(k * L, L)] for k in range(HALF)]
        c3c = [c3_v[b, pl.ds(k * L, L)] for k in range(HALF)]

        def row_body(r, out_acc):
            rr = j * L + r
            v = [buf[rr, pl.ds(k * L, L)] for k in range(2 * HALF)]
            sa = v[0] * v[0]
            sb = v[HALF] * v[HALF]
            for k in range(1, HALF):
                sa = sa + v[k] * v[k]
                sb = sb + v[HALF + k] * v[HALF + k]
            ra = _rsqrt16(jnp.full((L,), jnp.sum(sa)))
            rb = _rsqrt16(jnp.full((L,), jnp.sum(sb)))
            acc = jnp.abs(v[0] * ra * c1c[0] - v[HALF] * rb * c2c[0] + c3c[0])
            for k in range(1, HALF):
                t = v[k] * ra * c1c[k] - v[HALF + k] * rb * c2c[k] + c3c[k]
                acc = acc + jnp.abs(t)
            score = GAMMA - jnp.sum(acc)
            return jnp.where(lanes == r, score, out_acc)

        out16 = lax.fori_loop(0, L, row_body, zf)
        out_v[pl.ds(pl.multiple_of(gidx * L, L), L)] = out16

    def chunk_pair(i, carry):
        b, g = carry
        for half, (buf, sem) in enumerate(((rb0, sem0), (rb1, sem1))):
            cc = i * 2 + half
            gather_chunk(cc, buf, sem).wait()
            for j in range(GPC):
                process_group(buf, j, b, cc * GPC + j)
                g = g + 1
                wrap = (g >= GPB).astype(jnp.int32)
                g = g * (1 - wrap)
                b = b + wrap
            nc = cc + 2

            @pl.when(nc < NCHUNK)
            def _():
                gather_chunk(nc, buf, sem).start()
        return (b, g)

    lax.fori_loop(0, NCHUNK // 2, chunk_pair,
                  (jnp.int32(0), jnp.int32(0)))

    for b in range(BPW):
        pltpu.sync_copy(out_v.at[pl.ds(b * NPAD, N)], out_hbm.at[b0 + b])


def kernel(entity_embedding, relation_embedding, positive_sample, negative_sample):
    rel_idx = positive_sample[:, 1].astype(jnp.int32)
    tail_idx = positive_sample[:, 2].astype(jnp.int32)
    mesh = plsc.VectorSubcoreMesh(core_axis_name="c", subcore_axis_name="s",
                                  num_cores=NC, num_subcores=NS)
    f = pl.kernel(
        _body,
        out_type=jax.ShapeDtypeStruct((B, N), jnp.float32),
        mesh=mesh,
        compiler_params=pltpu.CompilerParams(use_tc_tiling_on_sc=False,
                                             needs_layout_passes=False),
        scratch_types=[
            pltpu.VMEM((BPW,), jnp.int32),
            pltpu.VMEM((BPW,), jnp.int32),
            pltpu.VMEM((BPW, ED), jnp.float32),
            pltpu.VMEM((BPW, RD), jnp.float32),
            pltpu.VMEM((BPW, D), jnp.float32),
            pltpu.VMEM((BPW, D), jnp.float32),
            pltpu.VMEM((BPW, D), jnp.float32),
            pltpu.VMEM((ROWS_PW,), jnp.int32),
            pltpu.VMEM((ROWS_PW,), jnp.float32),
            pltpu.VMEM((CHUNK, ED), jnp.float32),
            pltpu.VMEM((CHUNK, ED), jnp.float32),
            pltpu.SemaphoreType.DMA,
            pltpu.SemaphoreType.DMA,
            pltpu.SemaphoreType.DMA,
        ],
    )
    return f(entity_embedding, relation_embedding, rel_idx, tail_idx,
             negative_sample)


# speedup vs baseline: 3.4221x; 1.0068x over previous
import jax
import jax.numpy as jnp
from jax import lax
from jax.experimental import pallas as pl
from jax.experimental.pallas import tpu as pltpu
from jax.experimental.pallas import tpu_sc as plsc

NC, NS, L = 2, 16, 16
NW = NC * NS
B = 1024
N = 200
NPAD = 208
GPB = NPAD // L
BPW = B // NW
D = 128
ED = 2 * D
RD = 3 * D
CHUNK = 64
GPC = CHUNK // L
ROWS_PW = BPW * NPAD
NCHUNK = ROWS_PW // CHUNK
GAMMA = 12.0


def _rsqrt16(x):
    xi = plsc.bitcast(x, jnp.int32)
    yi = jnp.int32(0x5F3759DF) - lax.shift_right_logical(xi, 1)
    y = plsc.bitcast(yi, jnp.float32)
    for _ in range(2):
        y = y * (1.5 - 0.5 * x * y * y)
    return y


def _body(ent_hbm, rel_hbm, relidx_hbm, tailidx_hbm, neg_hbm, out_hbm,
          tidx_v, ridx_v, tail_v, rel_v, c1_v, c2_v, c3_v, nidx_v, out_v,
          rb0, rb1, sem0, sem1, gsem):
    wid = lax.axis_index("s") * NC + lax.axis_index("c")
    b0 = pl.multiple_of(wid * BPW, BPW)
    lanes = lax.iota(jnp.int32, L)
    zf = jnp.zeros((L,), jnp.float32)

    pltpu.sync_copy(tailidx_hbm.at[pl.ds(b0, BPW)], tidx_v)
    pltpu.sync_copy(relidx_hbm.at[pl.ds(b0, BPW)], ridx_v)
    pltpu.make_async_copy(ent_hbm.at[tidx_v], tail_v, gsem).start()
    pltpu.make_async_copy(ent_hbm.at[tidx_v], tail_v, gsem).wait()
    pltpu.make_async_copy(rel_hbm.at[ridx_v], rel_v, gsem).start()
    pltpu.make_async_copy(rel_hbm.at[ridx_v], rel_v, gsem).wait()

    zi = jnp.zeros((L,), jnp.int32)
    for b in range(BPW):
        nidx_v[pl.ds(b * NPAD + NPAD - L, L)] = zi
        pltpu.sync_copy(neg_hbm.at[b0 + b], nidx_v.at[pl.ds(b * NPAD, N)])

    for gb in range(BPW // L):
        rows = jnp.int32(gb * L) + lanes

        def ssq_body(d, carry):
            aa, ab = carry
            dv = jnp.full((L,), d, jnp.int32)
            va = plsc.load_gather(tail_v, [rows, dv])
            vb = plsc.load_gather(tail_v, [rows, dv + D])
            return (aa + va * va, ab + vb * vb)

        aa, ab = lax.fori_loop(0, D, ssq_body, (zf, zf))
        ra = _rsqrt16(aa)
        rb = _rsqrt16(ab)

        def cstore_body(d, _):
            dv = jnp.full((L,), d, jnp.int32)
            va = plsc.load_gather(tail_v, [rows, dv])
            vb = plsc.load_gather(tail_v, [rows, dv + D])
            vm = plsc.load_gather(rel_v, [rows, dv + D])
            c2 = va * ra
            plsc.store_scatter(c1_v, [rows, dv], vb * rb + 1.0)
            plsc.store_scatter(c2_v, [rows, dv], c2)
            plsc.store_scatter(c3_v, [rows, dv], vm - c2)
            return 0

        lax.fori_loop(0, D, cstore_body, 0)

    def gather_chunk(c, buf, sem):
        off = pl.multiple_of(c * CHUNK, CHUNK)
        return pltpu.make_async_copy(
            ent_hbm.at[nidx_v.at[pl.ds(off, CHUNK)]], buf, sem)

    gather_chunk(0, rb0, sem0).start()
    gather_chunk(1, rb1, sem1).start()

    HALF = D // L

    def process_group(buf, j, b, gidx):
        def row_body(r, out_acc):
            rr = j * L + r
            v = [buf[rr, pl.ds(k * L, L)] for k in range(2 * HALF)]
            sa = v[0] * v[0]
            sb = v[HALF] * v[HALF]
            for k in range(1, HALF):
                sa = sa + v[k] * v[k]
                sb = sb + v[HALF + k] * v[HALF + k]
            ra = _rsqrt16(jnp.full((L,), jnp.sum(sa)))
            rb = _rsqrt16(jnp.full((L,), jnp.sum(sb)))
            acc = zf
            for k in range(HALF):
                sl = pl.ds(k * L, L)
                t = v[k] * ra * c1_v[b, sl] - v[HALF + k] * rb * c2_v[b, sl] \
                    + c3_v[b, sl]
                acc = acc + jnp.abs(t)
            score = GAMMA - jnp.sum(acc)
            return jnp.where(lanes == r, score, out_acc)

        out16 = lax.fori_loop(0, L, row_body, zf)
        out_v[pl.ds(pl.multiple_of(gidx * L, L), L)] = out16

    def chunk_pair(i, carry):
        b, g = carry
        for half, (buf, sem) in enumerate(((rb0, sem0), (rb1, sem1))):
            cc = i * 2 + half
            gather_chunk(cc, buf, sem).wait()
            for j in range(GPC):
                process_group(buf, j, b, cc * GPC + j)
                g = g + 1
                wrap = (g >= GPB).astype(jnp.int32)
                g = g * (1 - wrap)
                b = b + wrap
            nc = cc + 2

            @pl.when(nc < NCHUNK)
            def _():
                gather_chunk(nc, buf, sem).start()
        return (b, g)

    lax.fori_loop(0, NCHUNK // 2, chunk_pair,
                  (jnp.int32(0), jnp.int32(0)))

    for b in range(BPW):
        pltpu.sync_copy(out_v.at[pl.ds(b * NPAD, N)], out_hbm.at[b0 + b])


def kernel(entity_embedding, relation_embedding, positive_sample, negative_sample):
    rel_idx = positive_sample[:, 1].astype(jnp.int32)
    tail_idx = positive_sample[:, 2].astype(jnp.int32)
    mesh = plsc.VectorSubcoreMesh(core_axis_name="c", subcore_axis_name="s",
                                  num_cores=NC, num_subcores=NS)
    f = pl.kernel(
        _body,
        out_type=jax.ShapeDtypeStruct((B, N), jnp.float32),
        mesh=mesh,
        compiler_params=pltpu.CompilerParams(use_tc_tiling_on_sc=False,
                                             needs_layout_passes=False),
        scratch_types=[
            pltpu.VMEM((BPW,), jnp.int32),
            pltpu.VMEM((BPW,), jnp.int32),
            pltpu.VMEM((BPW, ED), jnp.float32),
            pltpu.VMEM((BPW, RD), jnp.float32),
            pltpu.VMEM((BPW, D), jnp.float32),
            pltpu.VMEM((BPW, D), jnp.float32),
            pltpu.VMEM((BPW, D), jnp.float32),
            pltpu.VMEM((ROWS_PW,), jnp.int32),
            pltpu.VMEM((ROWS_PW,), jnp.float32),
            pltpu.VMEM((CHUNK, ED), jnp.float32),
            pltpu.VMEM((CHUNK, ED), jnp.float32),
            pltpu.SemaphoreType.DMA,
            pltpu.SemaphoreType.DMA,
            pltpu.SemaphoreType.DMA,
        ],
    )
    return f(entity_embedding, relation_embedding, rel_idx, tail_idx,
             negative_sample)


# speedup vs baseline: 3.4330x; 1.0032x over previous
import jax
import jax.numpy as jnp
from jax import lax
from jax.experimental import pallas as pl
from jax.experimental.pallas import tpu as pltpu
from jax.experimental.pallas import tpu_sc as plsc

NC, NS, L = 2, 16, 16
NW = NC * NS
B = 1024
N = 200
NPAD = 208
GPB = NPAD // L
BPW = B // NW
D = 128
ED = 2 * D
RD = 3 * D
CHUNK = 64
GPC = CHUNK // L
ROWS_PW = BPW * NPAD
NCHUNK = ROWS_PW // CHUNK
GAMMA = 12.0


def _rsqrt16(x):
    xi = plsc.bitcast(x, jnp.int32)
    yi = jnp.int32(0x5F3759DF) - lax.shift_right_logical(xi, 1)
    y = plsc.bitcast(yi, jnp.float32)
    for _ in range(2):
        y = y * (1.5 - 0.5 * x * y * y)
    return y


def _body(ent_hbm, rel_hbm, relidx_hbm, tailidx_hbm, neg_hbm, out_hbm,
          tidx_v, ridx_v, tail_v, rel_v, c1_v, c2_v, c3_v, nidx_v, out_v,
          rb0, rb1, sem0, sem1, gsem):
    wid = lax.axis_index("s") * NC + lax.axis_index("c")
    b0 = pl.multiple_of(wid * BPW, BPW)
    lanes = lax.iota(jnp.int32, L)
    zf = jnp.zeros((L,), jnp.float32)

    pltpu.sync_copy(tailidx_hbm.at[pl.ds(b0, BPW)], tidx_v)
    pltpu.sync_copy(relidx_hbm.at[pl.ds(b0, BPW)], ridx_v)
    pltpu.make_async_copy(ent_hbm.at[tidx_v], tail_v, gsem).start()
    pltpu.make_async_copy(ent_hbm.at[tidx_v], tail_v, gsem).wait()
    pltpu.make_async_copy(rel_hbm.at[ridx_v], rel_v, gsem).start()
    pltpu.make_async_copy(rel_hbm.at[ridx_v], rel_v, gsem).wait()

    zi = jnp.zeros((L,), jnp.int32)
    for b in range(BPW):
        nidx_v[pl.ds(b * NPAD + NPAD - L, L)] = zi
        pltpu.sync_copy(neg_hbm.at[b0 + b], nidx_v.at[pl.ds(b * NPAD, N)])

    for gb in range(BPW // L):
        rows = jnp.int32(gb * L) + lanes

        def ssq_body(d, carry):
            aa, ab = carry
            dv = jnp.full((L,), d, jnp.int32)
            va = plsc.load_gather(tail_v, [rows, dv])
            vb = plsc.load_gather(tail_v, [rows, dv + D])
            return (aa + va * va, ab + vb * vb)

        aa, ab = lax.fori_loop(0, D, ssq_body, (zf, zf))
        ra = _rsqrt16(aa)
        rb = _rsqrt16(ab)

        def cstore_body(d, _):
            dv = jnp.full((L,), d, jnp.int32)
            va = plsc.load_gather(tail_v, [rows, dv])
            vb = plsc.load_gather(tail_v, [rows, dv + D])
            vm = plsc.load_gather(rel_v, [rows, dv + D])
            c2 = va * ra
            plsc.store_scatter(c1_v, [rows, dv], vb * rb + 1.0)
            plsc.store_scatter(c2_v, [rows, dv], c2)
            plsc.store_scatter(c3_v, [rows, dv], vm - c2)
            return 0

        lax.fori_loop(0, D, cstore_body, 0)

    def gather_chunk(c, buf, sem):
        off = pl.multiple_of(c * CHUNK, CHUNK)
        return pltpu.make_async_copy(
            ent_hbm.at[nidx_v.at[pl.ds(off, CHUNK)]], buf, sem)

    gather_chunk(0, rb0, sem0).start()
    gather_chunk(1, rb1, sem1).start()

    HALF = D // L

    def chunk_pair(i, carry):
        bg = carry
        for half, (buf, sem) in enumerate(((rb0, sem0), (rb1, sem1))):
            cc = i * 2 + half
            gather_chunk(cc, buf, sem).wait()

            def group_body(j, bg):
                b, g = bg
                gidx = cc * GPC + j
                c1c = [c1_v[b, pl.ds(k * L, L)] for k in range(HALF)]
                c2c = [c2_v[b, pl.ds(k * L, L)] for k in range(HALF)]
                c3c = [c3_v[b, pl.ds(k * L, L)] for k in range(HALF)]

                sa16 = zf
                sb16 = zf
                for r in range(L):
                    rr = j * L + r
                    sa = None
                    sb = None
                    for k in range(HALF):
                        va = buf[rr, pl.ds(k * L, L)]
                        vb = buf[rr, pl.ds((HALF + k) * L, L)]
                        sa = va * va if sa is None else sa + va * va
                        sb = vb * vb if sb is None else sb + vb * vb
                    sa16 = jnp.where(lanes == r, jnp.sum(sa), sa16)
                    sb16 = jnp.where(lanes == r, jnp.sum(sb), sb16)
                ra16 = _rsqrt16(sa16)
                rb16 = _rsqrt16(sb16)

                out16 = zf
                for r in range(L):
                    rr = j * L + r
                    ra = ra16[r]
                    rb = rb16[r]
                    acc = None
                    for k in range(HALF):
                        va = buf[rr, pl.ds(k * L, L)]
                        vb = buf[rr, pl.ds((HALF + k) * L, L)]
                        t = jnp.abs(va * ra * c1c[k] - vb * rb * c2c[k]
                                    + c3c[k])
                        acc = t if acc is None else acc + t
                    out16 = jnp.where(lanes == r, GAMMA - jnp.sum(acc), out16)
                out_v[pl.ds(pl.multiple_of(gidx * L, L), L)] = out16

                g = g + 1
                wrap = (g >= GPB).astype(jnp.int32)
                return (b + wrap, g * (1 - wrap))

            bg = lax.fori_loop(0, GPC, group_body, bg)
            nc = cc + 2

            @pl.when(nc < NCHUNK)
            def _():
                gather_chunk(nc, buf, sem).start()
        return bg

    lax.fori_loop(0, NCHUNK // 2, chunk_pair,
                  (jnp.int32(0), jnp.int32(0)))

    for b in range(BPW):
        pltpu.sync_copy(out_v.at[pl.ds(b * NPAD, N)], out_hbm.at[b0 + b])


def kernel(entity_embedding, relation_embedding, positive_sample, negative_sample):
    rel_idx = positive_sample[:, 1].astype(jnp.int32)
    tail_idx = positive_sample[:, 2].astype(jnp.int32)
    mesh = plsc.VectorSubcoreMesh(core_axis_name="c", subcore_axis_name="s",
                                  num_cores=NC, num_subcores=NS)
    f = pl.kernel(
        _body,
        out_type=jax.ShapeDtypeStruct((B, N), jnp.float32),
        mesh=mesh,
        compiler_params=pltpu.CompilerParams(use_tc_tiling_on_sc=False,
                                             needs_layout_passes=False),
        scratch_types=[
            pltpu.VMEM((BPW,), jnp.int32),
            pltpu.VMEM((BPW,), jnp.int32),
            pltpu.VMEM((BPW, ED), jnp.float32),
            pltpu.VMEM((BPW, RD), jnp.float32),
            pltpu.VMEM((BPW, D), jnp.float32),
            pltpu.VMEM((BPW, D), jnp.float32),
            pltpu.VMEM((BPW, D), jnp.float32),
            pltpu.VMEM((ROWS_PW,), jnp.int32),
            pltpu.VMEM((ROWS_PW,), jnp.float32),
            pltpu.VMEM((CHUNK, ED), jnp.float32),
            pltpu.VMEM((CHUNK, ED), jnp.float32),
            pltpu.SemaphoreType.DMA,
            pltpu.SemaphoreType.DMA,
            pltpu.SemaphoreType.DMA,
        ],
    )
    return f(entity_embedding, relation_embedding, rel_idx, tail_idx,
             negative_sample)


# speedup vs baseline: 3.9459x; 1.1494x over previous
import jax
import jax.numpy as jnp
from jax import lax
from jax.experimental import pallas as pl
from jax.experimental.pallas import tpu as pltpu
from jax.experimental.pallas import tpu_sc as plsc

NC, NS, L = 2, 16, 16
NW = NC * NS
B = 1024
N = 200
NPAD = 208
GPB = NPAD // L
BPW = B // NW
D = 128
ED = 2 * D
RD = 3 * D
CHUNK = 64
GPC = CHUNK // L
ROWS_PW = BPW * NPAD
NCHUNK = ROWS_PW // CHUNK
GAMMA = 12.0


def _rsqrt16(x):
    xi = plsc.bitcast(x, jnp.int32)
    yi = jnp.int32(0x5F3759DF) - lax.shift_right_logical(xi, 1)
    y = plsc.bitcast(yi, jnp.float32)
    for _ in range(2):
        y = y * (1.5 - 0.5 * x * y * y)
    return y


def _body(ent_hbm, rel_hbm, relidx_hbm, tailidx_hbm, neg_hbm, out_hbm,
          tidx_v, ridx_v, tail_v, rel_v, c1_v, c2_v, c3_v, nidx_v, out_v,
          rb0, rb1, sem0, sem1, gsem):
    wid = lax.axis_index("s") * NC + lax.axis_index("c")
    b0 = pl.multiple_of(wid * BPW, BPW)
    lanes = lax.iota(jnp.int32, L)
    zf = jnp.zeros((L,), jnp.float32)

    pltpu.sync_copy(tailidx_hbm, tidx_v)
    pltpu.sync_copy(relidx_hbm, ridx_v)
    tsl = tidx_v.at[pl.ds(b0, BPW)]
    rsl = ridx_v.at[pl.ds(b0, BPW)]
    pltpu.make_async_copy(ent_hbm.at[tsl], tail_v, gsem).start()
    pltpu.make_async_copy(ent_hbm.at[tsl], tail_v, gsem).wait()
    pltpu.make_async_copy(rel_hbm.at[rsl], rel_v, gsem).start()
    pltpu.make_async_copy(rel_hbm.at[rsl], rel_v, gsem).wait()

    pltpu.sync_copy(
        neg_hbm.at[pl.ds(pl.multiple_of(wid * ROWS_PW, ROWS_PW), ROWS_PW)],
        nidx_v)

    for gb in range(BPW // L):
        rows = jnp.int32(gb * L) + lanes

        def ssq_body(d, carry):
            aa, ab = carry
            dv = jnp.full((L,), d, jnp.int32)
            va = plsc.load_gather(tail_v, [rows, dv])
            vb = plsc.load_gather(tail_v, [rows, dv + D])
            return (aa + va * va, ab + vb * vb)

        aa, ab = lax.fori_loop(0, D, ssq_body, (zf, zf))
        ra = _rsqrt16(aa)
        rb = _rsqrt16(ab)

        def cstore_body(d, _):
            dv = jnp.full((L,), d, jnp.int32)
            va = plsc.load_gather(tail_v, [rows, dv])
            vb = plsc.load_gather(tail_v, [rows, dv + D])
            vm = plsc.load_gather(rel_v, [rows, dv + D])
            c2 = va * ra
            plsc.store_scatter(c1_v, [rows, dv], vb * rb + 1.0)
            plsc.store_scatter(c2_v, [rows, dv], c2)
            plsc.store_scatter(c3_v, [rows, dv], vm - c2)
            return 0

        lax.fori_loop(0, D, cstore_body, 0)

    def gather_chunk(c, buf, sem):
        off = pl.multiple_of(c * CHUNK, CHUNK)
        return pltpu.make_async_copy(
            ent_hbm.at[nidx_v.at[pl.ds(off, CHUNK)]], buf, sem)

    gather_chunk(0, rb0, sem0).start()
    gather_chunk(1, rb1, sem1).start()

    HALF = D // L

    def chunk_pair(i, carry):
        bg = carry
        for half, (buf, sem) in enumerate(((rb0, sem0), (rb1, sem1))):
            cc = i * 2 + half
            gather_chunk(cc, buf, sem).wait()

            def group_body(j, bg):
                b, g = bg
                gidx = cc * GPC + j
                c1c = [c1_v[b, pl.ds(k * L, L)] for k in range(HALF)]
                c2c = [c2_v[b, pl.ds(k * L, L)] for k in range(HALF)]
                c3c = [c3_v[b, pl.ds(k * L, L)] for k in range(HALF)]

                sa16 = zf
                sb16 = zf
                for r in range(L):
                    rr = j * L + r
                    sa = None
                    sb = None
                    for k in range(HALF):
                        va = buf[rr, pl.ds(k * L, L)]
                        vb = buf[rr, pl.ds((HALF + k) * L, L)]
                        sa = va * va if sa is None else sa + va * va
                        sb = vb * vb if sb is None else sb + vb * vb
                    sa16 = jnp.where(lanes == r, jnp.sum(sa), sa16)
                    sb16 = jnp.where(lanes == r, jnp.sum(sb), sb16)
                ra16 = _rsqrt16(sa16)
                rb16 = _rsqrt16(sb16)

                out16 = zf
                for r in range(L):
                    rr = j * L + r
                    ra = ra16[r]
                    rb = rb16[r]
                    acc = None
                    for k in range(HALF):
                        va = buf[rr, pl.ds(k * L, L)]
                        vb = buf[rr, pl.ds((HALF + k) * L, L)]
                        t = jnp.abs(va * ra * c1c[k] - vb * rb * c2c[k]
                                    + c3c[k])
                        acc = t if acc is None else acc + t
                    out16 = jnp.where(lanes == r, GAMMA - jnp.sum(acc), out16)
                out_v[pl.ds(pl.multiple_of(b * N + g * L, 8), L)] = out16

                g = g + 1
                wrap = (g >= GPB).astype(jnp.int32)
                return (b + wrap, g * (1 - wrap))

            bg = lax.fori_loop(0, GPC, group_body, bg)
            nc = cc + 2

            @pl.when(nc < NCHUNK)
            def _():
                gather_chunk(nc, buf, sem).start()
        return bg

    lax.fori_loop(0, NCHUNK // 2, chunk_pair,
                  (jnp.int32(0), jnp.int32(0)))

    npw = BPW * N
    pltpu.sync_copy(out_v.at[pl.ds(0, npw)],
                    out_hbm.at[pl.ds(pl.multiple_of(wid * npw, npw), npw)])


def kernel(entity_embedding, relation_embedding, positive_sample, negative_sample):
    rel_idx = positive_sample[:, 1].astype(jnp.int32)
    tail_idx = positive_sample[:, 2].astype(jnp.int32)
    neg_pad = jnp.pad(negative_sample, ((0, 0), (0, NPAD - N))).reshape(-1)
    mesh = plsc.VectorSubcoreMesh(core_axis_name="c", subcore_axis_name="s",
                                  num_cores=NC, num_subcores=NS)
    f = pl.kernel(
        _body,
        out_type=jax.ShapeDtypeStruct((B * N,), jnp.float32),
        mesh=mesh,
        compiler_params=pltpu.CompilerParams(use_tc_tiling_on_sc=True,
                                             needs_layout_passes=False),
        scratch_types=[
            pltpu.VMEM((B,), jnp.int32),
            pltpu.VMEM((B,), jnp.int32),
            pltpu.VMEM((BPW, ED), jnp.float32),
            pltpu.VMEM((BPW, RD), jnp.float32),
            pltpu.VMEM((BPW, D), jnp.float32),
            pltpu.VMEM((BPW, D), jnp.float32),
            pltpu.VMEM((BPW, D), jnp.float32),
            pltpu.VMEM((ROWS_PW,), jnp.int32),
            pltpu.VMEM((ROWS_PW,), jnp.float32),
            pltpu.VMEM((CHUNK, ED), jnp.float32),
            pltpu.VMEM((CHUNK, ED), jnp.float32),
            pltpu.SemaphoreType.DMA,
            pltpu.SemaphoreType.DMA,
            pltpu.SemaphoreType.DMA,
        ],
    )
    out = f(entity_embedding, relation_embedding, rel_idx, tail_idx, neg_pad)
    return out.reshape(B, N)


# speedup vs baseline: 4.0073x; 1.0156x over previous
import jax
import jax.numpy as jnp
from jax import lax
from jax.experimental import pallas as pl
from jax.experimental.pallas import tpu as pltpu
from jax.experimental.pallas import tpu_sc as plsc

NC, NS, L = 2, 16, 16
NW = NC * NS
B = 1024
N = 200
NPAD = 208
GPB = NPAD // L
BPW = B // NW
D = 128
ED = 2 * D
RD = 3 * D
CHUNK = 64
GPC = CHUNK // L
ROWS_PW = BPW * NPAD
NCHUNK = ROWS_PW // CHUNK
GAMMA = 12.0


def _rsqrt16(x):
    xi = plsc.bitcast(x, jnp.int32)
    yi = jnp.int32(0x5F3759DF) - lax.shift_right_logical(xi, 1)
    y = plsc.bitcast(yi, jnp.float32)
    for _ in range(2):
        y = y * (1.5 - 0.5 * x * y * y)
    return y


def _body(ent_hbm, rel_hbm, relidx_hbm, tailidx_hbm, neg_hbm, out_hbm,
          tidx_v, ridx_v, tail_v, rel_v, c1_v, c2_v, c3_v, nidx_v, out_v,
          rb0, rb1, sem0, sem1, gsem):
    wid = lax.axis_index("s") * NC + lax.axis_index("c")
    b0 = pl.multiple_of(wid * BPW, BPW)
    lanes = lax.iota(jnp.int32, L)
    zf = jnp.zeros((L,), jnp.float32)

    pltpu.sync_copy(tailidx_hbm, tidx_v)
    pltpu.sync_copy(relidx_hbm, ridx_v)
    tsl = tidx_v.at[pl.ds(b0, BPW)]
    rsl = ridx_v.at[pl.ds(b0, BPW)]
    pltpu.make_async_copy(ent_hbm.at[tsl], tail_v, gsem).start()
    pltpu.make_async_copy(ent_hbm.at[tsl], tail_v, gsem).wait()
    pltpu.make_async_copy(rel_hbm.at[rsl], rel_v, gsem).start()
    pltpu.make_async_copy(rel_hbm.at[rsl], rel_v, gsem).wait()

    pltpu.sync_copy(
        neg_hbm.at[pl.ds(pl.multiple_of(wid * ROWS_PW, ROWS_PW), ROWS_PW)],
        nidx_v)

    for gb in range(BPW // L):
        rows = jnp.int32(gb * L) + lanes

        def ssq_body(d, carry):
            aa, ab = carry
            dv = jnp.full((L,), d, jnp.int32)
            va = plsc.load_gather(tail_v, [rows, dv])
            vb = plsc.load_gather(tail_v, [rows, dv + D])
            return (aa + va * va, ab + vb * vb)

        aa, ab = lax.fori_loop(0, D, ssq_body, (zf, zf))
        ra = _rsqrt16(aa)
        rb = _rsqrt16(ab)

        def cstore_body(d, _):
            dv = jnp.full((L,), d, jnp.int32)
            va = plsc.load_gather(tail_v, [rows, dv])
            vb = plsc.load_gather(tail_v, [rows, dv + D])
            vm = plsc.load_gather(rel_v, [rows, dv + D])
            c2 = va * ra
            plsc.store_scatter(c1_v, [rows, dv], vb * rb + 1.0)
            plsc.store_scatter(c2_v, [rows, dv], c2)
            plsc.store_scatter(c3_v, [rows, dv], vm - c2)
            return 0

        lax.fori_loop(0, D, cstore_body, 0)

    def gather_chunk(c, buf, sem):
        off = pl.multiple_of(c * CHUNK, CHUNK)
        return pltpu.make_async_copy(
            ent_hbm.at[nidx_v.at[pl.ds(off, CHUNK)]], buf, sem)

    gather_chunk(0, rb0, sem0).start()
    gather_chunk(1, rb1, sem1).start()

    HALF = D // L

    def chunk_pair(i, carry):
        bg = carry
        for half, (buf, sem) in enumerate(((rb0, sem0), (rb1, sem1))):
            cc = i * 2 + half
            gather_chunk(cc, buf, sem).wait()

            def group_body(j, bg):
                b, g = bg
                gidx = cc * GPC + j
                c1c = [c1_v[b, pl.ds(k * L, L)] for k in range(HALF)]
                c2c = [c2_v[b, pl.ds(k * L, L)] for k in range(HALF)]
                c3c = [c3_v[b, pl.ds(k * L, L)] for k in range(HALF)]

                @plsc.parallel_loop(0, L, unroll=2, carry=(zf, zf))
                def _norms(r, c):
                    sa16, sb16 = c
                    rr = j * L + r
                    sa = None
                    sb = None
                    for k in range(HALF):
                        va = buf[rr, pl.ds(k * L, L)]
                        vb = buf[rr, pl.ds((HALF + k) * L, L)]
                        sa = va * va if sa is None else sa + va * va
                        sb = vb * vb if sb is None else sb + vb * vb
                    sa16 = jnp.where(lanes == r, jnp.sum(sa), sa16)
                    sb16 = jnp.where(lanes == r, jnp.sum(sb), sb16)
                    return sa16, sb16

                sa16, sb16 = _norms
                ra16 = _rsqrt16(sa16)
                rb16 = _rsqrt16(sb16)

                @plsc.parallel_loop(0, L, unroll=2, carry=zf)
                def out16(r, out_acc):
                    rr = j * L + r
                    rsp = jnp.full((L,), r, jnp.int32)
                    ra = ra16.at[rsp].get(mode="promise_in_bounds")
                    rb = rb16.at[rsp].get(mode="promise_in_bounds")
                    acc = None
                    for k in range(HALF):
                        va = buf[rr, pl.ds(k * L, L)]
                        vb = buf[rr, pl.ds((HALF + k) * L, L)]
                        t = jnp.abs(va * ra * c1c[k] - vb * rb * c2c[k]
                                    + c3c[k])
                        acc = t if acc is None else acc + t
                    return jnp.where(lanes == r, GAMMA - jnp.sum(acc),
                                     out_acc)
                out_v[pl.ds(pl.multiple_of(b * N + g * L, 8), L)] = out16

                g = g + 1
                wrap = (g >= GPB).astype(jnp.int32)
                return (b + wrap, g * (1 - wrap))

            bg = lax.fori_loop(0, GPC, group_body, bg)
            nc = cc + 2

            @pl.when(nc < NCHUNK)
            def _():
                gather_chunk(nc, buf, sem).start()
        return bg

    lax.fori_loop(0, NCHUNK // 2, chunk_pair,
                  (jnp.int32(0), jnp.int32(0)))

    npw = BPW * N
    pltpu.sync_copy(out_v.at[pl.ds(0, npw)],
                    out_hbm.at[pl.ds(pl.multiple_of(wid * npw, npw), npw)])


def kernel(entity_embedding, relation_embedding, positive_sample, negative_sample):
    rel_idx = positive_sample[:, 1].astype(jnp.int32)
    tail_idx = positive_sample[:, 2].astype(jnp.int32)
    neg_pad = jnp.pad(negative_sample, ((0, 0), (0, NPAD - N))).reshape(-1)
    mesh = plsc.VectorSubcoreMesh(core_axis_name="c", subcore_axis_name="s",
                                  num_cores=NC, num_subcores=NS)
    f = pl.kernel(
        _body,
        out_type=jax.ShapeDtypeStruct((B * N,), jnp.float32),
        mesh=mesh,
        compiler_params=pltpu.CompilerParams(use_tc_tiling_on_sc=True,
                                             needs_layout_passes=False),
        scratch_types=[
            pltpu.VMEM((B,), jnp.int32),
            pltpu.VMEM((B,), jnp.int32),
            pltpu.VMEM((BPW, ED), jnp.float32),
            pltpu.VMEM((BPW, RD), jnp.float32),
            pltpu.VMEM((BPW, D), jnp.float32),
            pltpu.VMEM((BPW, D), jnp.float32),
            pltpu.VMEM((BPW, D), jnp.float32),
            pltpu.VMEM((ROWS_PW,), jnp.int32),
            pltpu.VMEM((ROWS_PW,), jnp.float32),
            pltpu.VMEM((CHUNK, ED), jnp.float32),
            pltpu.VMEM((CHUNK, ED), jnp.float32),
            pltpu.SemaphoreType.DMA,
            pltpu.SemaphoreType.DMA,
            pltpu.SemaphoreType.DMA,
        ],
    )
    out = f(entity_embedding, relation_embedding, rel_idx, tail_idx, neg_pad)
    return out.reshape(B, N)
